# Initial kernel scaffold; baseline (speedup 1.0000x reference)
import jax
import jax.numpy as jnp
from jax.experimental import pallas as pl


def kernel(x, t, params):
    raise NotImplementedError("write your pallas kernel here")


# speedup vs baseline: 1.5381x; 1.5381x over previous
import functools

import jax
import jax.numpy as jnp
import numpy as np
from jax.experimental import pallas as pl

_TED = 128
_BIG = 1e10


def _fps_body(npoint, x_ref, y_ref, z_ref, cent_ref):
    B, N = x_ref.shape
    iota = jax.lax.broadcasted_iota(jnp.int32, (B, N), 1)
    lane_np = jax.lax.broadcasted_iota(jnp.int32, (B, npoint), 1)
    xs = x_ref[...]
    ys = y_ref[...]
    zs = z_ref[...]

    def step(i, carry):
        dist, far = carry
        m32 = (lane_np == i).astype(jnp.int32)
        cur = cent_ref[...]
        cent_ref[...] = cur + (far - cur) * m32
        sel = iota == far
        cx = jnp.sum(jnp.where(sel, xs, 0.0), axis=1, keepdims=True)
        cy = jnp.sum(jnp.where(sel, ys, 0.0), axis=1, keepdims=True)
        cz = jnp.sum(jnp.where(sel, zs, 0.0), axis=1, keepdims=True)
        dx = xs - cx
        dy = ys - cy
        dz = zs - cz
        d = dx * dx + dy * dy + dz * dz
        dist = jnp.minimum(dist, d)
        m = jnp.max(dist, axis=1, keepdims=True)
        far_new = jnp.min(
            jnp.where(dist == m, iota, N), axis=1, keepdims=True
        ).astype(jnp.int32)
        return dist, far_new

    dist0 = xs * 0.0 + _BIG
    far0 = (xs[:, 0:1] * 0.0).astype(jnp.int32)
    jax.lax.fori_loop(0, npoint, step, (dist0, far0))


def _fps(xyz, npoint):
    B, N, _ = xyz.shape
    xt = jnp.transpose(xyz, (0, 2, 1))
    return pl.pallas_call(
        functools.partial(_fps_body, npoint),
        out_shape=jax.ShapeDtypeStruct((B, npoint), jnp.int32),
    )(xt[:, 0], xt[:, 1], xt[:, 2])


def _pair_d(q, p):
    qx = q[:, 0:1]
    qy = q[:, 1:2]
    qz = q[:, 2:3]
    qn = qx * qx + qy * qy + qz * qz
    px = p[0:1, :]
    py = p[1:2, :]
    pz = p[2:3, :]
    pn = px * px + py * py + pz * pz
    cross = jnp.dot(q, p, preferred_element_type=jnp.float32)
    return qn + pn - 2.0 * cross


def _ballq_body(r2, nsample, q_ref, p_ref, out_ref):
    q = q_ref[0]
    p = p_ref[0]
    S = q.shape[0]
    N = p.shape[1]
    d = _pair_d(q, p)
    iota = jax.lax.broadcasted_iota(jnp.int32, (S, N), 1)
    lane_k = jax.lax.broadcasted_iota(jnp.int32, (S, nsample), 1)
    idxm = jnp.where(d > r2, N, iota)
    first = jnp.min(idxm, axis=1, keepdims=True)

    def step(s, cur):
        cand = jnp.min(jnp.where(idxm > cur, idxm, N), axis=1, keepdims=True)
        sel = jnp.where(cand == N, first, cand).astype(jnp.int32)
        m32 = (lane_k == s).astype(jnp.int32)
        acc = out_ref[0]
        out_ref[0] = acc + (sel - acc) * m32
        return jnp.where(cand == N, cur, cand).astype(jnp.int32)

    cur0 = (q[:, 0:1] * 0.0).astype(jnp.int32) - 1
    jax.lax.fori_loop(0, nsample, step, cur0)


def _ballq(radius, nsample, xyz, new_xyz):
    B, N, _ = xyz.shape
    S = new_xyz.shape[1]
    pt = jnp.transpose(xyz, (0, 2, 1))
    return pl.pallas_call(
        functools.partial(_ballq_body, radius * radius, nsample),
        grid=(B,),
        in_specs=[
            pl.BlockSpec((1, S, 3), lambda b: (b, 0, 0)),
            pl.BlockSpec((1, 3, N), lambda b: (b, 0, 0)),
        ],
        out_specs=pl.BlockSpec((1, S, nsample), lambda b: (b, 0, 0)),
        out_shape=jax.ShapeDtypeStruct((B, S, nsample), jnp.int32),
    )(new_xyz, pt)


def _knn3_body(q_ref, p_ref, idx_ref, w_ref):
    q = q_ref[0]
    p = p_ref[0]
    S = q.shape[0]
    N = p.shape[1]
    d = _pair_d(q, p)
    iota = jax.lax.broadcasted_iota(jnp.int32, (S, N), 1)
    work = d
    js = []
    ms = []
    for _ in range(3):
        m = jnp.min(work, axis=1, keepdims=True)
        j = jnp.min(jnp.where(work == m, iota, N), axis=1, keepdims=True)
        js.append(j.astype(jnp.int32))
        ms.append(m)
        work = jnp.where(iota == j, _BIG, work)
    r0 = 1.0 / (ms[0] + 1e-8)
    r1 = 1.0 / (ms[1] + 1e-8)
    r2 = 1.0 / (ms[2] + 1e-8)
    rs = r0 + r1 + r2
    idx_ref[0] = jnp.concatenate(js, axis=1)
    w_ref[0] = jnp.concatenate([r0 / rs, r1 / rs, r2 / rs], axis=1)


def _knn3(xyz1, xyz2):
    B, N1, _ = xyz1.shape
    N2 = xyz2.shape[1]
    pt = jnp.transpose(xyz2, (0, 2, 1))
    return pl.pallas_call(
        _knn3_body,
        grid=(B,),
        in_specs=[
            pl.BlockSpec((1, N1, 3), lambda b: (b, 0, 0)),
            pl.BlockSpec((1, 3, N2), lambda b: (b, 0, 0)),
        ],
        out_specs=[
            pl.BlockSpec((1, N1, 3), lambda b: (b, 0, 0)),
            pl.BlockSpec((1, N1, 3), lambda b: (b, 0, 0)),
        ],
        out_shape=[
            jax.ShapeDtypeStruct((B, N1, 3), jnp.int32),
            jax.ShapeDtypeStruct((B, N1, 3), jnp.float32),
        ],
    )(xyz1, pt)


def _index_pts(points, idx):
    return jax.vmap(lambda p, i: p[i])(points, idx)


def _time_emb(t, dim):
    half = dim // 2
    e = np.log(10000.0) / (half - 1)
    e = jnp.exp(jnp.arange(half) * -e)
    e = t[:, None] * e[None, :]
    return jnp.concatenate([jnp.sin(e), jnp.cos(e)], axis=-1)


def _cconv(x, te, p):
    tproj = jax.nn.gelu(te) @ p['tw'].T + p['tb']
    x = x + tproj[:, :, None]
    x = jnp.einsum('oc,bcl->bol', p['cw'], x) + p['cb'][None, :, None]
    m = jnp.mean(x, axis=(0, 2), keepdims=True)
    v = jnp.var(x, axis=(0, 2), keepdims=True)
    x = (x - m) / jnp.sqrt(v + 1e-5)
    x = x * p['g'][None, :, None] + p['bb'][None, :, None]
    return jax.nn.gelu(x)


def _sa(xyz, points, te, npoint, radius, nsample, convs):
    if npoint is None:
        if points is not None:
            pp = jnp.concatenate(
                [xyz, jnp.transpose(points, (0, 2, 1))], axis=-1)
        else:
            pp = xyz
        pp = jnp.transpose(pp, (0, 2, 1))
        for p in convs:
            pp = _cconv(pp, te, p)
        return None, jnp.max(pp, axis=2)[:, :, None]
    fi = _fps(xyz, npoint)
    new_xyz = _index_pts(xyz, fi)
    idx = _ballq(radius, nsample, xyz, new_xyz)
    g = _index_pts(xyz, idx) - new_xyz[:, :, None, :]
    if points is not None:
        gp = _index_pts(jnp.transpose(points, (0, 2, 1)), idx)
        h = jnp.concatenate([g, gp], axis=-1)
    else:
        h = g
    h = jnp.transpose(h, (0, 3, 1, 2))
    for p in convs:
        B, C, S, K = h.shape
        h = _cconv(h.reshape(B, C, S * K), te, p).reshape(B, -1, S, K)
    return new_xyz, jnp.max(h, axis=3)


def _fp(xyz1, xyz2, points1, points2, te, convs):
    B, N, _ = xyz1.shape
    S = points2.shape[2]
    if S == 1:
        interp = jnp.broadcast_to(points2, (B, points2.shape[1], N))
    else:
        idx, w = _knn3(xyz1, xyz2)
        interp = jnp.sum(
            _index_pts(jnp.transpose(points2, (0, 2, 1)), idx)
            * w[..., None], axis=2)
        interp = jnp.transpose(interp, (0, 2, 1))
    x = interp if points1 is None else jnp.concatenate([points1, interp], axis=1)
    for p in convs:
        x = _cconv(x, te, p)
    return x


def kernel(x, t, params):
    te = _time_emb(t, _TED)
    l1x, l1p = _sa(x, None, te, 512, 0.2, 32, params['sa1'])
    l2x, l2p = _sa(l1x, l1p, te, 128, 0.4, 64, params['sa2'])
    l3x, l3p = _sa(l2x, l2p, te, None, None, None, params['sa3'])
    l2p = _fp(l2x, l3x, l2p, l3p, te, params['fp3'])
    l1p = _fp(l1x, l2x, l1p, l2p, te, params['fp2'])
    l0p = _fp(x, l1x, None, l1p, te, params['fp1'])
    return (jnp.einsum('oc,bcl->bol', params['head_w'], l0p)
            + params['head_b'][None, :, None])


# speedup vs baseline: 1.7279x; 1.1234x over previous
import functools

import jax
import jax.numpy as jnp
import numpy as np
from jax.experimental import pallas as pl

_TED = 128
_BIG = 1e10


def _fps_body(npoint, x_ref, y_ref, z_ref, cent_ref):
    B, N = x_ref.shape
    iota = jax.lax.broadcasted_iota(jnp.int32, (B, N), 1)
    lane_np = jax.lax.broadcasted_iota(jnp.int32, (B, npoint), 1)
    xs = x_ref[...]
    ys = y_ref[...]
    zs = z_ref[...]

    def step(i, carry):
        dist, far = carry
        m32 = (lane_np == i).astype(jnp.int32)
        cur = cent_ref[...]
        cent_ref[...] = cur + (far - cur) * m32
        sel = iota == far
        cx = jnp.sum(jnp.where(sel, xs, 0.0), axis=1, keepdims=True)
        cy = jnp.sum(jnp.where(sel, ys, 0.0), axis=1, keepdims=True)
        cz = jnp.sum(jnp.where(sel, zs, 0.0), axis=1, keepdims=True)
        dx = xs - cx
        dy = ys - cy
        dz = zs - cz
        d = dx * dx + dy * dy + dz * dz
        dist = jnp.minimum(dist, d)
        m = jnp.max(dist, axis=1, keepdims=True)
        far_new = jnp.min(
            jnp.where(dist == m, iota, N), axis=1, keepdims=True
        ).astype(jnp.int32)
        return dist, far_new

    dist0 = xs * 0.0 + _BIG
    far0 = (xs[:, 0:1] * 0.0).astype(jnp.int32)
    jax.lax.fori_loop(0, npoint, step, (dist0, far0))


def _fps(xyz, npoint):
    B, N, _ = xyz.shape
    xt = jnp.transpose(xyz, (0, 2, 1))
    return pl.pallas_call(
        functools.partial(_fps_body, npoint),
        out_shape=jax.ShapeDtypeStruct((B, npoint), jnp.int32),
    )(xt[:, 0], xt[:, 1], xt[:, 2])


def _pair_d(q, p):
    qx = q[:, 0:1]
    qy = q[:, 1:2]
    qz = q[:, 2:3]
    qn = qx * qx + qy * qy + qz * qz
    px = p[0:1, :]
    py = p[1:2, :]
    pz = p[2:3, :]
    pn = px * px + py * py + pz * pz
    cross = jnp.dot(q, p, preferred_element_type=jnp.float32)
    return qn + pn - 2.0 * cross


def _ballq_body(r2, nsample, q_ref, p_ref, out_ref):
    q = q_ref[0]
    p = p_ref[0]
    S = q.shape[0]
    N = p.shape[1]
    d = _pair_d(q, p)
    iota = jax.lax.broadcasted_iota(jnp.int32, (S, N), 1)
    lane_k = jax.lax.broadcasted_iota(jnp.int32, (S, nsample), 1)
    idxm = jnp.where(d > r2, N, iota)
    first = jnp.min(idxm, axis=1, keepdims=True)

    def step(s, cur):
        cand = jnp.min(jnp.where(idxm > cur, idxm, N), axis=1, keepdims=True)
        sel = jnp.where(cand == N, first, cand).astype(jnp.int32)
        m32 = (lane_k == s).astype(jnp.int32)
        acc = out_ref[0]
        out_ref[0] = acc + (sel - acc) * m32
        return jnp.where(cand == N, cur, cand).astype(jnp.int32)

    cur0 = (q[:, 0:1] * 0.0).astype(jnp.int32) - 1
    jax.lax.fori_loop(0, nsample, step, cur0)


def _ballq(radius, nsample, xyz, new_xyz):
    B, N, _ = xyz.shape
    S = new_xyz.shape[1]
    pt = jnp.transpose(xyz, (0, 2, 1))
    return pl.pallas_call(
        functools.partial(_ballq_body, radius * radius, nsample),
        grid=(B,),
        in_specs=[
            pl.BlockSpec((1, S, 3), lambda b: (b, 0, 0)),
            pl.BlockSpec((1, 3, N), lambda b: (b, 0, 0)),
        ],
        out_specs=pl.BlockSpec((1, S, nsample), lambda b: (b, 0, 0)),
        out_shape=jax.ShapeDtypeStruct((B, S, nsample), jnp.int32),
    )(new_xyz, pt)


def _knn3_body(q_ref, p_ref, idx_ref, w_ref):
    q = q_ref[0]
    p = p_ref[0]
    S = q.shape[0]
    N = p.shape[1]
    d = _pair_d(q, p)
    iota = jax.lax.broadcasted_iota(jnp.int32, (S, N), 1)
    work = d
    js = []
    ms = []
    for _ in range(3):
        m = jnp.min(work, axis=1, keepdims=True)
        j = jnp.min(jnp.where(work == m, iota, N), axis=1, keepdims=True)
        js.append(j.astype(jnp.int32))
        ms.append(m)
        work = jnp.where(iota == j, _BIG, work)
    r0 = 1.0 / (ms[0] + 1e-8)
    r1 = 1.0 / (ms[1] + 1e-8)
    r2 = 1.0 / (ms[2] + 1e-8)
    rs = r0 + r1 + r2
    idx_ref[0] = jnp.concatenate(js, axis=1)
    w_ref[0] = jnp.concatenate([r0 / rs, r1 / rs, r2 / rs], axis=1)


def _knn3(xyz1, xyz2):
    B, N1, _ = xyz1.shape
    N2 = xyz2.shape[1]
    pt = jnp.transpose(xyz2, (0, 2, 1))
    return pl.pallas_call(
        _knn3_body,
        grid=(B,),
        in_specs=[
            pl.BlockSpec((1, N1, 3), lambda b: (b, 0, 0)),
            pl.BlockSpec((1, 3, N2), lambda b: (b, 0, 0)),
        ],
        out_specs=[
            pl.BlockSpec((1, N1, 3), lambda b: (b, 0, 0)),
            pl.BlockSpec((1, N1, 3), lambda b: (b, 0, 0)),
        ],
        out_shape=[
            jax.ShapeDtypeStruct((B, N1, 3), jnp.int32),
            jax.ShapeDtypeStruct((B, N1, 3), jnp.float32),
        ],
    )(xyz1, pt)


def _index_pts(points, idx):
    return jax.vmap(lambda p, i: p[i])(points, idx)


def _time_emb(t, dim):
    half = dim // 2
    e = np.log(10000.0) / (half - 1)
    e = jnp.exp(jnp.arange(half) * -e)
    e = t[:, None] * e[None, :]
    return jnp.concatenate([jnp.sin(e), jnp.cos(e)], axis=-1)


def _cconv(x, te, p):
    tproj = jax.nn.gelu(te) @ p['tw'].T + p['tb']
    x = x + tproj[:, :, None]
    x = jnp.einsum('oc,bcl->bol', p['cw'], x) + p['cb'][None, :, None]
    m = jnp.mean(x, axis=(0, 2), keepdims=True)
    v = jnp.var(x, axis=(0, 2), keepdims=True)
    x = (x - m) / jnp.sqrt(v + 1e-5)
    x = x * p['g'][None, :, None] + p['bb'][None, :, None]
    return jax.nn.gelu(x)


def _sa(xyz, points, te, npoint, radius, nsample, convs):
    if npoint is None:
        if points is not None:
            pp = jnp.concatenate(
                [xyz, jnp.transpose(points, (0, 2, 1))], axis=-1)
        else:
            pp = xyz
        pp = jnp.transpose(pp, (0, 2, 1))
        for p in convs:
            pp = _cconv(pp, te, p)
        return None, jnp.max(pp, axis=2)[:, :, None]
    fi = _fps(xyz, npoint)
    new_xyz = _index_pts(xyz, fi)
    idx = _ballq(radius, nsample, xyz, new_xyz)
    g = _index_pts(xyz, idx) - new_xyz[:, :, None, :]
    if points is not None:
        gp = _index_pts(jnp.transpose(points, (0, 2, 1)), idx)
        h = jnp.concatenate([g, gp], axis=-1)
    else:
        h = g
    h = jnp.transpose(h, (0, 3, 1, 2))
    for p in convs:
        B, C, S, K = h.shape
        h = _cconv(h.reshape(B, C, S * K), te, p).reshape(B, -1, S, K)
    return new_xyz, jnp.max(h, axis=3)


def _fp(xyz1, xyz2, points1, points2, te, convs):
    B, N, _ = xyz1.shape
    S = points2.shape[2]
    if S == 1:
        interp = jnp.broadcast_to(points2, (B, points2.shape[1], N))
    else:
        idx, w = _knn3(xyz1, xyz2)
        oh = jax.nn.one_hot(idx, S, dtype=jnp.float32)
        wm = jnp.sum(oh * w[..., None], axis=2)
        interp = jnp.einsum('bcs,bns->bcn', points2, wm,
                            precision=jax.lax.Precision.HIGHEST)
    x = interp if points1 is None else jnp.concatenate([points1, interp], axis=1)
    for p in convs:
        x = _cconv(x, te, p)
    return x


def kernel(x, t, params):
    te = _time_emb(t, _TED)
    l1x, l1p = _sa(x, None, te, 512, 0.2, 32, params['sa1'])
    l2x, l2p = _sa(l1x, l1p, te, 128, 0.4, 64, params['sa2'])
    l3x, l3p = _sa(l2x, l2p, te, None, None, None, params['sa3'])
    l2p = _fp(l2x, l3x, l2p, l3p, te, params['fp3'])
    l1p = _fp(l1x, l2x, l1p, l2p, te, params['fp2'])
    l0p = _fp(x, l1x, None, l1p, te, params['fp1'])
    return (jnp.einsum('oc,bcl->bol', params['head_w'], l0p)
            + params['head_b'][None, :, None])
